# baseline (speedup 1.0000x reference)
import functools

import jax
import jax.numpy as jnp
from jax import lax
from jax.experimental import pallas as pl
from jax.experimental.pallas import tpu as pltpu
from jax.experimental.pallas import tpu_sc as plsc

N = 10000
NPAD = 10240
E = 320000
EPAD = 327680
NW = 32
EB = 80
LB = 128
D_IN = 128
DH = 35
DHP = 48
DO = 2
DOP = 16
RB = 10240
NSLICE = NPAD // 16


def _mesh():
    return plsc.VectorSubcoreMesh(core_axis_name="c", subcore_axis_name="s")


_SC_PARAMS = pltpu.CompilerParams(use_tc_tiling_on_sc=False,
                                  needs_layout_passes=False)


def _make_msg(width, fuse_deg, stage_tab=True, nbuf=4):
    niter = EB // nbuf

    def body(*args):
        sems = args[len(args) - 2 * nbuf:]
        gsem = sems[:nbuf]
        ssem = sems[nbuf:]
        if fuse_deg:
            (rows_hbm, src_hbm, dst_hbm, out_hbm, dinv_hbm,
             sidx, didx, rows, zbuf, ones_v, degv, dv, rv,
             tab, acc, deg_sh, dsem) = args[:17]
        elif stage_tab:
            (rows_hbm, src_hbm, dst_hbm, out_hbm,
             sidx, didx, rows, zbuf,
             tab, acc) = args[:10]
        else:
            (rows_hbm, src_hbm, dst_hbm, out_hbm,
             sidx, didx, rows, zbuf,
             acc) = args[:9]
            tab = rows_hbm
        cid = lax.axis_index("c")
        sid = lax.axis_index("s")
        tile = cid * 16 + sid

        def zb(i, _):
            for k in range(width // 16):
                zbuf[i, pl.ds(k * 16, 16)] = jnp.zeros((16,), jnp.float32)
            return 0

        lax.fori_loop(0, LB, zb, 0)
        for r in range(NSLICE // LB):
            pltpu.sync_copy(zbuf,
                            acc.at[pl.ds(sid * NSLICE + r * LB, LB)])

        if fuse_deg:
            def ob(i, _):
                ones_v[pl.ds(i * 16, 16)] = jnp.ones((16,), jnp.float32)
                return 0

            lax.fori_loop(0, LB // 16, ob, 0)

            def zdeg(i, _):
                degv[pl.ds(i * 16, 16)] = jnp.zeros((16,), jnp.float32)
                return 0

            lax.fori_loop(0, NSLICE // 16, zdeg, 0)
            pltpu.sync_copy(degv, deg_sh.at[pl.ds(sid * NSLICE, NSLICE)])
            plsc.subcore_barrier()

            def chunk(c, _):
                def fire(j, _):
                    pltpu.async_copy(ones_v,
                                     deg_sh.at[didx.at[c * 20 + j]],
                                     dsem, add=True)
                    return 0

                lax.fori_loop(0, 20, fire, 0)

                def drain(j, _):
                    pltpu.make_async_copy(ones_v, deg_sh.at[didx.at[0]],
                                          dsem).wait()
                    return 0

                lax.fori_loop(0, 20, drain, 0)
                return 0

            def half(c2, _):
                pltpu.sync_copy(dst_hbm.at[2 * sid + c2], didx)
                lax.fori_loop(0, EB // 20, chunk, 0)
                return 0

            lax.fori_loop(0, 2, half, 0)
            plsc.subcore_barrier()

            pltpu.sync_copy(deg_sh.at[pl.ds(sid * NSLICE, NSLICE)], degv)

            def nwt(i, _):
                d = degv[pl.ds(i * 16, 16)] + 1.0
                ib = plsc.bitcast(d, jnp.int32)
                y = plsc.bitcast(jnp.int32(0x5F3759DF) - (ib >> 1),
                                 jnp.float32)
                for _u in range(3):
                    y = y * (1.5 - 0.5 * d * y * y)
                dv[pl.ds(i * 16, 16)] = y
                return 0

            lax.fori_loop(0, NSLICE // 16, nwt, 0)
            pltpu.sync_copy(dv, dinv_hbm.at[cid,
                                            pl.ds(sid * NSLICE, NSLICE)])

            def schunk(c, _):
                base = sid * NSLICE + c * LB
                pltpu.sync_copy(rows_hbm.at[pl.ds(base, LB)], rv)

                def srow(r, _):
                    dvec = plsc.load_gather(
                        dv, [jnp.full((16,), c * LB + r, jnp.int32)])
                    for k in range(width // 16):
                        rv[r, pl.ds(k * 16, 16)] = (
                            rv[r, pl.ds(k * 16, 16)] * dvec)
                    return 0

                lax.fori_loop(0, LB, srow, 0)
                pltpu.sync_copy(rv, tab.at[pl.ds(base, LB)])
                return 0

            lax.fori_loop(0, NSLICE // LB, schunk, 0)
        elif stage_tab:
            pltpu.sync_copy(rows_hbm.at[pl.ds(sid * NSLICE, NSLICE)],
                            tab.at[pl.ds(sid * NSLICE, NSLICE)])
        plsc.subcore_barrier()

        pltpu.sync_copy(src_hbm.at[tile], sidx)
        pltpu.sync_copy(dst_hbm.at[tile], didx)

        def g_start(b, jj):
            pltpu.async_copy(tab.at[sidx.at[jj]], rows.at[b], gsem[b])

        def g_wait(b, jj):
            pltpu.make_async_copy(tab.at[sidx.at[jj]], rows.at[b],
                                  gsem[b]).wait()

        def s_start(b, jj):
            pltpu.async_copy(rows.at[b], acc.at[didx.at[jj]], ssem[b],
                             add=True)

        def s_wait(b, jj):
            pltpu.make_async_copy(rows.at[b], acc.at[didx.at[jj]],
                                  ssem[b]).wait()

        for b in range(nbuf):
            g_start(b, b)

        def it(i, _):
            for b in range(nbuf):
                g_wait(b, i * nbuf + b)
                s_start(b, i * nbuf + b)

            @pl.when(i < niter - 1)
            def _():
                for b in range(nbuf):
                    s_wait(b, i * nbuf + b)
                    g_start(b, (i + 1) * nbuf + b)

            return 0

        lax.fori_loop(0, niter, it, 0)
        for b in range(nbuf):
            s_wait(b, b)
        plsc.subcore_barrier()
        pltpu.sync_copy(acc.at[pl.ds(sid * NSLICE, NSLICE)],
                        out_hbm.at[cid, pl.ds(sid * NSLICE, NSLICE)])

    if fuse_deg:
        out_type = [
            jax.ShapeDtypeStruct((2, NPAD, width), jnp.float32),
            jax.ShapeDtypeStruct((2, NPAD), jnp.float32),
        ]
        scratch = [
            pltpu.VMEM((EB, LB), jnp.int32),
            pltpu.VMEM((EB, LB), jnp.int32),
            pltpu.VMEM((nbuf, LB, width), jnp.float32),
            pltpu.VMEM((LB, width), jnp.float32),
            pltpu.VMEM((LB,), jnp.float32),
            pltpu.VMEM((NSLICE,), jnp.float32),
            pltpu.VMEM((NSLICE,), jnp.float32),
            pltpu.VMEM((LB, width), jnp.float32),
            pltpu.VMEM_SHARED((NPAD, width), jnp.float32),
            pltpu.VMEM_SHARED((NPAD, width), jnp.float32),
            pltpu.VMEM_SHARED((NPAD,), jnp.float32),
        ] + [pltpu.SemaphoreType.DMA] * (1 + 2 * nbuf)
    else:
        out_type = jax.ShapeDtypeStruct((2, NPAD, width), jnp.float32)
        scratch = [
            pltpu.VMEM((EB, LB), jnp.int32),
            pltpu.VMEM((EB, LB), jnp.int32),
            pltpu.VMEM((nbuf, LB, width), jnp.float32),
            pltpu.VMEM((LB, width), jnp.float32),
        ] + ([pltpu.VMEM_SHARED((NPAD, width), jnp.float32)]
             if stage_tab else []) + [
            pltpu.VMEM_SHARED((NPAD, width), jnp.float32),
        ] + [pltpu.SemaphoreType.DMA] * (2 * nbuf)
    return functools.partial(
        pl.kernel,
        out_type=out_type,
        mesh=_mesh(),
        scratch_types=scratch,
        compiler_params=_SC_PARAMS,
    )(body)


def _tc_b(x_ref, w1_ref, h_ref):
    h_ref[...] = jnp.dot(x_ref[...], w1_ref[...],
                         preferred_element_type=jnp.float32)


def _tc_d(p_ref, h1_ref, dinv_ref, b1_ref, w2_ref, gs_ref):
    p = p_ref[...]
    dinv = dinv_ref[...]
    s = p[0] + p[1] + h1_ref[...] * dinv
    out1 = s * dinv + b1_ref[...]
    h2 = jnp.maximum(out1, 0.0)
    g = jnp.dot(h2, w2_ref[...], preferred_element_type=jnp.float32)
    gs_ref[...] = g * dinv


def _tc_f(q_ref, gs_ref, dinv_ref, b2_ref, out_ref):
    q = q_ref[...]
    out2 = (q[0] + q[1] + gs_ref[...]) * dinv_ref[...] + b2_ref[...]
    a = out2[:, 0:1]
    b = out2[:, 1:2]
    m = jnp.maximum(a, b)
    lse = m + jnp.log(jnp.exp(a - m) + jnp.exp(b - m))
    out_ref[...] = out2[:, 0:2] - lse


def kernel(x, edge_index, W1, b1, W2, b2):
    src = edge_index[0].astype(jnp.int32)
    dst = edge_index[1].astype(jnp.int32)
    pad = jnp.full((EPAD - E,), NPAD - 1, jnp.int32)
    src3 = jnp.concatenate([src, pad]).reshape(NW, EB, LB)
    dst3 = jnp.concatenate([dst, pad]).reshape(NW, EB, LB)
    xp = jnp.pad(x, ((0, NPAD - N), (0, 0)))
    W1p = jnp.pad(W1, ((0, 0), (0, DHP - DH)))
    b1p = jnp.pad(b1, (0, DHP - DH)).reshape(1, DHP)
    W2p = jnp.pad(W2, ((0, DHP - DH), (0, DOP - DO)))
    b2p = jnp.pad(b2, (0, DOP - DO)).reshape(1, DOP)

    grid = NPAD // RB
    h1 = pl.pallas_call(
        _tc_b,
        grid=(grid,),
        in_specs=[
            pl.BlockSpec((RB, D_IN), lambda i: (i, 0)),
            pl.BlockSpec((D_IN, DHP), lambda i: (0, 0)),
        ],
        out_specs=pl.BlockSpec((RB, DHP), lambda i: (i, 0)),
        out_shape=jax.ShapeDtypeStruct((NPAD, DHP), jnp.float32),
    )(xp, W1p)

    p1, dinv2 = _make_msg(DHP, True, nbuf=5)(h1, src3, dst3)
    dinv = dinv2[0].reshape(NPAD, 1)

    gs = pl.pallas_call(
        _tc_d,
        grid=(grid,),
        in_specs=[
            pl.BlockSpec((2, RB, DHP), lambda i: (0, i, 0)),
            pl.BlockSpec((RB, DHP), lambda i: (i, 0)),
            pl.BlockSpec((RB, 1), lambda i: (i, 0)),
            pl.BlockSpec((1, DHP), lambda i: (0, 0)),
            pl.BlockSpec((DHP, DOP), lambda i: (0, 0)),
        ],
        out_specs=pl.BlockSpec((RB, DOP), lambda i: (i, 0)),
        out_shape=jax.ShapeDtypeStruct((NPAD, DOP), jnp.float32),
    )(p1, h1, dinv, b1p, W2p)

    p2 = _make_msg(DOP, False, stage_tab=True, nbuf=8)(gs, src3, dst3)

    out = pl.pallas_call(
        _tc_f,
        grid=(grid,),
        in_specs=[
            pl.BlockSpec((2, RB, DOP), lambda i: (0, i, 0)),
            pl.BlockSpec((RB, DOP), lambda i: (i, 0)),
            pl.BlockSpec((RB, 1), lambda i: (i, 0)),
            pl.BlockSpec((1, DOP), lambda i: (0, 0)),
        ],
        out_specs=pl.BlockSpec((RB, DO), lambda i: (i, 0)),
        out_shape=jax.ShapeDtypeStruct((NPAD, DO), jnp.float32),
    )(p2, gs, dinv, b2p)

    return out[:N]


# speedup vs baseline: 1.0096x; 1.0096x over previous
import functools

import jax
import jax.numpy as jnp
from jax import lax
from jax.experimental import pallas as pl
from jax.experimental.pallas import tpu as pltpu
from jax.experimental.pallas import tpu_sc as plsc

N = 10000
NPAD = 10240
E = 320000
EPAD = 327680
NW = 32
EB = 80
LB = 128
D_IN = 128
DH = 35
DHP = 48
DO = 2
DOP = 16
RB = 2048
NSLICE = NPAD // 16


def _mesh():
    return plsc.VectorSubcoreMesh(core_axis_name="c", subcore_axis_name="s")


_SC_PARAMS = pltpu.CompilerParams(use_tc_tiling_on_sc=False,
                                  needs_layout_passes=False)


def _make_msg(width, fuse_deg, stage_tab=True, nbuf=4):
    niter = EB // nbuf

    def body(*args):
        sems = args[len(args) - 2 * nbuf:]
        gsem = sems[:nbuf]
        ssem = sems[nbuf:]
        if fuse_deg:
            (rows_hbm, src_hbm, dst_hbm, out_hbm, dinv_hbm,
             sidx, didx, rows, zbuf, ones_v, degv, dv, rv,
             tab, acc, deg_sh, dsem) = args[:17]
        elif stage_tab:
            (rows_hbm, src_hbm, dst_hbm, out_hbm,
             sidx, didx, rows, zbuf,
             tab, acc) = args[:10]
        else:
            (rows_hbm, src_hbm, dst_hbm, out_hbm,
             sidx, didx, rows, zbuf,
             acc) = args[:9]
            tab = rows_hbm
        cid = lax.axis_index("c")
        sid = lax.axis_index("s")
        tile = cid * 16 + sid

        def zb(i, _):
            for k in range(width // 16):
                zbuf[i, pl.ds(k * 16, 16)] = jnp.zeros((16,), jnp.float32)
            return 0

        lax.fori_loop(0, LB, zb, 0)
        for r in range(NSLICE // LB):
            pltpu.sync_copy(zbuf,
                            acc.at[pl.ds(sid * NSLICE + r * LB, LB)])

        if fuse_deg:
            def ob(i, _):
                ones_v[pl.ds(i * 16, 16)] = jnp.ones((16,), jnp.float32)
                return 0

            lax.fori_loop(0, LB // 16, ob, 0)

            def zdeg(i, _):
                degv[pl.ds(i * 16, 16)] = jnp.zeros((16,), jnp.float32)
                return 0

            lax.fori_loop(0, NSLICE // 16, zdeg, 0)
            pltpu.sync_copy(degv, deg_sh.at[pl.ds(sid * NSLICE, NSLICE)])
            plsc.subcore_barrier()

            def chunk(c, _):
                def fire(j, _):
                    pltpu.async_copy(ones_v,
                                     deg_sh.at[didx.at[c * 40 + j]],
                                     dsem, add=True)
                    return 0

                lax.fori_loop(0, 40, fire, 0)

                def drain(j, _):
                    pltpu.make_async_copy(ones_v, deg_sh.at[didx.at[0]],
                                          dsem).wait()
                    return 0

                lax.fori_loop(0, 40, drain, 0)
                return 0

            def half(c2, _):
                pltpu.sync_copy(dst_hbm.at[2 * sid + c2], didx)
                lax.fori_loop(0, EB // 40, chunk, 0)
                return 0

            lax.fori_loop(0, 2, half, 0)
            plsc.subcore_barrier()

            pltpu.sync_copy(deg_sh.at[pl.ds(sid * NSLICE, NSLICE)], degv)

            def nwt(i, _):
                d = degv[pl.ds(i * 16, 16)] + 1.0
                ib = plsc.bitcast(d, jnp.int32)
                y = plsc.bitcast(jnp.int32(0x5F3759DF) - (ib >> 1),
                                 jnp.float32)
                for _u in range(3):
                    y = y * (1.5 - 0.5 * d * y * y)
                dv[pl.ds(i * 16, 16)] = y
                return 0

            lax.fori_loop(0, NSLICE // 16, nwt, 0)
            pltpu.sync_copy(dv, dinv_hbm.at[cid,
                                            pl.ds(sid * NSLICE, NSLICE)])

            def schunk(c, _):
                base = sid * NSLICE + c * LB
                pltpu.sync_copy(rows_hbm.at[pl.ds(base, LB)], rv)

                def srow(r, _):
                    dvec = plsc.load_gather(
                        dv, [jnp.full((16,), c * LB + r, jnp.int32)])
                    for k in range(width // 16):
                        rv[r, pl.ds(k * 16, 16)] = (
                            rv[r, pl.ds(k * 16, 16)] * dvec)
                    return 0

                lax.fori_loop(0, LB, srow, 0)
                pltpu.sync_copy(rv, tab.at[pl.ds(base, LB)])
                return 0

            lax.fori_loop(0, NSLICE // LB, schunk, 0)
        elif stage_tab:
            pltpu.sync_copy(rows_hbm.at[pl.ds(sid * NSLICE, NSLICE)],
                            tab.at[pl.ds(sid * NSLICE, NSLICE)])
        plsc.subcore_barrier()

        pltpu.sync_copy(src_hbm.at[tile], sidx)
        pltpu.sync_copy(dst_hbm.at[tile], didx)

        def g_start(b, jj):
            pltpu.async_copy(tab.at[sidx.at[jj]], rows.at[b], gsem[b])

        def g_wait(b, jj):
            pltpu.make_async_copy(tab.at[sidx.at[jj]], rows.at[b],
                                  gsem[b]).wait()

        def s_start(b, jj):
            pltpu.async_copy(rows.at[b], acc.at[didx.at[jj]], ssem[b],
                             add=True)

        def s_wait(b, jj):
            pltpu.make_async_copy(rows.at[b], acc.at[didx.at[jj]],
                                  ssem[b]).wait()

        for b in range(nbuf):
            g_start(b, b)

        def it(i, _):
            for b in range(nbuf):
                g_wait(b, i * nbuf + b)
                s_start(b, i * nbuf + b)

            @pl.when(i < niter - 1)
            def _():
                for b in range(nbuf):
                    s_wait(b, i * nbuf + b)
                    g_start(b, (i + 1) * nbuf + b)

            return 0

        lax.fori_loop(0, niter, it, 0)
        for b in range(nbuf):
            s_wait(b, b)
        plsc.subcore_barrier()
        pltpu.sync_copy(acc.at[pl.ds(sid * NSLICE, NSLICE)],
                        out_hbm.at[cid, pl.ds(sid * NSLICE, NSLICE)])

    if fuse_deg:
        out_type = [
            jax.ShapeDtypeStruct((2, NPAD, width), jnp.float32),
            jax.ShapeDtypeStruct((2, NPAD), jnp.float32),
        ]
        scratch = [
            pltpu.VMEM((EB, LB), jnp.int32),
            pltpu.VMEM((EB, LB), jnp.int32),
            pltpu.VMEM((nbuf, LB, width), jnp.float32),
            pltpu.VMEM((LB, width), jnp.float32),
            pltpu.VMEM((LB,), jnp.float32),
            pltpu.VMEM((NSLICE,), jnp.float32),
            pltpu.VMEM((NSLICE,), jnp.float32),
            pltpu.VMEM((LB, width), jnp.float32),
            pltpu.VMEM_SHARED((NPAD, width), jnp.float32),
            pltpu.VMEM_SHARED((NPAD, width), jnp.float32),
            pltpu.VMEM_SHARED((NPAD,), jnp.float32),
        ] + [pltpu.SemaphoreType.DMA] * (1 + 2 * nbuf)
    else:
        out_type = jax.ShapeDtypeStruct((2, NPAD, width), jnp.float32)
        scratch = [
            pltpu.VMEM((EB, LB), jnp.int32),
            pltpu.VMEM((EB, LB), jnp.int32),
            pltpu.VMEM((nbuf, LB, width), jnp.float32),
            pltpu.VMEM((LB, width), jnp.float32),
        ] + ([pltpu.VMEM_SHARED((NPAD, width), jnp.float32)]
             if stage_tab else []) + [
            pltpu.VMEM_SHARED((NPAD, width), jnp.float32),
        ] + [pltpu.SemaphoreType.DMA] * (2 * nbuf)
    return functools.partial(
        pl.kernel,
        out_type=out_type,
        mesh=_mesh(),
        scratch_types=scratch,
        compiler_params=_SC_PARAMS,
    )(body)


def _tc_b(x_ref, w1_ref, h_ref):
    h_ref[...] = jnp.dot(x_ref[...], w1_ref[...],
                         preferred_element_type=jnp.float32)


def _tc_d(p_ref, h1_ref, dinv_ref, b1_ref, w2_ref, gs_ref):
    p = p_ref[...]
    dinv = dinv_ref[...]
    s = p[0] + p[1] + h1_ref[...] * dinv
    out1 = s * dinv + b1_ref[...]
    h2 = jnp.maximum(out1, 0.0)
    g = jnp.dot(h2, w2_ref[...], preferred_element_type=jnp.float32)
    gs_ref[...] = g * dinv


def _tc_f(q_ref, gs_ref, dinv_ref, b2_ref, out_ref):
    q = q_ref[...]
    out2 = (q[0] + q[1] + gs_ref[...]) * dinv_ref[...] + b2_ref[...]
    a = out2[:, 0:1]
    b = out2[:, 1:2]
    m = jnp.maximum(a, b)
    lse = m + jnp.log(jnp.exp(a - m) + jnp.exp(b - m))
    out_ref[...] = out2[:, 0:2] - lse


def kernel(x, edge_index, W1, b1, W2, b2):
    src = edge_index[0].astype(jnp.int32)
    dst = edge_index[1].astype(jnp.int32)
    pad = jnp.full((EPAD - E,), NPAD - 1, jnp.int32)
    src3 = jnp.concatenate([src, pad]).reshape(NW, EB, LB)
    dst3 = jnp.concatenate([dst, pad]).reshape(NW, EB, LB)
    xp = jnp.pad(x, ((0, NPAD - N), (0, 0)))
    W1p = jnp.pad(W1, ((0, 0), (0, DHP - DH)))
    b1p = jnp.pad(b1, (0, DHP - DH)).reshape(1, DHP)
    W2p = jnp.pad(W2, ((0, DHP - DH), (0, DOP - DO)))
    b2p = jnp.pad(b2, (0, DOP - DO)).reshape(1, DOP)

    grid = NPAD // RB
    h1 = pl.pallas_call(
        _tc_b,
        grid=(grid,),
        in_specs=[
            pl.BlockSpec((RB, D_IN), lambda i: (i, 0)),
            pl.BlockSpec((D_IN, DHP), lambda i: (0, 0)),
        ],
        out_specs=pl.BlockSpec((RB, DHP), lambda i: (i, 0)),
        out_shape=jax.ShapeDtypeStruct((NPAD, DHP), jnp.float32),
    )(xp, W1p)

    p1, dinv2 = _make_msg(DHP, True, nbuf=5)(h1, src3, dst3)
    dinv = dinv2[0].reshape(NPAD, 1)

    gs = pl.pallas_call(
        _tc_d,
        grid=(grid,),
        in_specs=[
            pl.BlockSpec((2, RB, DHP), lambda i: (0, i, 0)),
            pl.BlockSpec((RB, DHP), lambda i: (i, 0)),
            pl.BlockSpec((RB, 1), lambda i: (i, 0)),
            pl.BlockSpec((1, DHP), lambda i: (0, 0)),
            pl.BlockSpec((DHP, DOP), lambda i: (0, 0)),
        ],
        out_specs=pl.BlockSpec((RB, DOP), lambda i: (i, 0)),
        out_shape=jax.ShapeDtypeStruct((NPAD, DOP), jnp.float32),
    )(p1, h1, dinv, b1p, W2p)

    p2 = _make_msg(DOP, False, stage_tab=True, nbuf=8)(gs, src3, dst3)

    out = pl.pallas_call(
        _tc_f,
        grid=(grid,),
        in_specs=[
            pl.BlockSpec((2, RB, DOP), lambda i: (0, i, 0)),
            pl.BlockSpec((RB, DOP), lambda i: (i, 0)),
            pl.BlockSpec((RB, 1), lambda i: (i, 0)),
            pl.BlockSpec((1, DOP), lambda i: (0, 0)),
        ],
        out_specs=pl.BlockSpec((RB, DO), lambda i: (i, 0)),
        out_shape=jax.ShapeDtypeStruct((NPAD, DO), jnp.float32),
    )(p2, gs, dinv, b2p)

    return out[:N]


# speedup vs baseline: 1.0765x; 1.0663x over previous
import functools

import jax
import jax.numpy as jnp
from jax import lax
from jax.experimental import pallas as pl
from jax.experimental.pallas import tpu as pltpu
from jax.experimental.pallas import tpu_sc as plsc

N = 10000
NPAD = 10240
E = 320000
EPAD = 327680
NW = 32
EB = 80
LB = 128
D_IN = 128
DH = 35
DHP = 48
DO = 2
DOP = 16
RB = 2048
NSLICE = NPAD // 16


def _mesh():
    return plsc.VectorSubcoreMesh(core_axis_name="c", subcore_axis_name="s")


_SC_PARAMS = pltpu.CompilerParams(use_tc_tiling_on_sc=False,
                                  needs_layout_passes=False)


def _deg_body(dst_hbm, out_hbm, idx_v, ones_v, zbuf_v, deg_sh, dsem):
    cid = lax.axis_index("c")
    sid = lax.axis_index("s")
    tile = cid * 16 + sid

    def zb(i, _):
        zbuf_v[pl.ds(i * 16, 16)] = jnp.zeros((16,), jnp.float32)
        return 0

    lax.fori_loop(0, NSLICE // 16, zb, 0)

    def ob(i, _):
        ones_v[pl.ds(i * 16, 16)] = jnp.ones((16,), jnp.float32)
        return 0

    lax.fori_loop(0, LB // 16, ob, 0)
    pltpu.sync_copy(zbuf_v, deg_sh.at[pl.ds(sid * NSLICE, NSLICE)])
    plsc.subcore_barrier()
    pltpu.sync_copy(dst_hbm.at[tile], idx_v)

    def chunk(c, _):
        def fire(j, _):
            pltpu.async_copy(ones_v, deg_sh.at[idx_v.at[c * 40 + j]],
                             dsem, add=True)
            return 0

        lax.fori_loop(0, 40, fire, 0)

        def drain(j, _):
            pltpu.make_async_copy(ones_v, deg_sh.at[idx_v.at[0]],
                                  dsem).wait()
            return 0

        lax.fori_loop(0, 40, drain, 0)
        return 0

    lax.fori_loop(0, EB // 40, chunk, 0)
    plsc.subcore_barrier()
    pltpu.sync_copy(deg_sh.at[pl.ds(sid * NSLICE, NSLICE)],
                    out_hbm.at[cid, pl.ds(sid * NSLICE, NSLICE)])


def _make_deg():
    return functools.partial(
        pl.kernel,
        out_type=jax.ShapeDtypeStruct((2, NPAD), jnp.float32),
        mesh=_mesh(),
        scratch_types=[
            pltpu.VMEM((EB, LB), jnp.int32),
            pltpu.VMEM((LB,), jnp.float32),
            pltpu.VMEM((NSLICE,), jnp.float32),
            pltpu.VMEM_SHARED((NPAD,), jnp.float32),
            pltpu.SemaphoreType.DMA,
        ],
        compiler_params=_SC_PARAMS,
    )(_deg_body)


def _make_msg(width, fuse_deg, stage_tab=True, nbuf=4):
    niter = EB // nbuf

    def body(*args):
        sems = args[len(args) - 2 * nbuf:]
        gsem = sems[:nbuf]
        ssem = sems[nbuf:]
        if fuse_deg:
            (rows_hbm, src_hbm, dst_hbm, degp_hbm, out_hbm, dinv_hbm,
             sidx, didx, rows, zbuf, degv, degv2, dv, rv,
             tab, acc) = args[:16]
        elif stage_tab:
            (rows_hbm, src_hbm, dst_hbm, out_hbm,
             sidx, didx, rows, zbuf,
             tab, acc) = args[:10]
        else:
            (rows_hbm, src_hbm, dst_hbm, out_hbm,
             sidx, didx, rows, zbuf,
             acc) = args[:9]
            tab = rows_hbm
        cid = lax.axis_index("c")
        sid = lax.axis_index("s")
        tile = cid * 16 + sid

        def zb(i, _):
            for k in range(width // 16):
                zbuf[i, pl.ds(k * 16, 16)] = jnp.zeros((16,), jnp.float32)
            return 0

        lax.fori_loop(0, LB, zb, 0)
        for r in range(NSLICE // LB):
            pltpu.sync_copy(zbuf,
                            acc.at[pl.ds(sid * NSLICE + r * LB, LB)])

        if fuse_deg:
            pltpu.sync_copy(degp_hbm.at[0, pl.ds(sid * NSLICE, NSLICE)],
                            degv)
            pltpu.sync_copy(degp_hbm.at[1, pl.ds(sid * NSLICE, NSLICE)],
                            degv2)

            def nwt(i, _):
                d = (degv[pl.ds(i * 16, 16)] + degv2[pl.ds(i * 16, 16)]
                     + 1.0)
                ib = plsc.bitcast(d, jnp.int32)
                y = plsc.bitcast(jnp.int32(0x5F3759DF) - (ib >> 1),
                                 jnp.float32)
                for _u in range(3):
                    y = y * (1.5 - 0.5 * d * y * y)
                dv[pl.ds(i * 16, 16)] = y
                return 0

            lax.fori_loop(0, NSLICE // 16, nwt, 0)
            pltpu.sync_copy(dv, dinv_hbm.at[cid,
                                            pl.ds(sid * NSLICE, NSLICE)])

            def schunk(c, _):
                base = sid * NSLICE + c * LB
                pltpu.sync_copy(rows_hbm.at[pl.ds(base, LB)], rv)

                def srow(r, _):
                    dvec = plsc.load_gather(
                        dv, [jnp.full((16,), c * LB + r, jnp.int32)])
                    for k in range(width // 16):
                        rv[r, pl.ds(k * 16, 16)] = (
                            rv[r, pl.ds(k * 16, 16)] * dvec)
                    return 0

                lax.fori_loop(0, LB, srow, 0)
                pltpu.sync_copy(rv, tab.at[pl.ds(base, LB)])
                return 0

            lax.fori_loop(0, NSLICE // LB, schunk, 0)
        elif stage_tab:
            pltpu.sync_copy(rows_hbm.at[pl.ds(sid * NSLICE, NSLICE)],
                            tab.at[pl.ds(sid * NSLICE, NSLICE)])
        plsc.subcore_barrier()

        pltpu.sync_copy(src_hbm.at[tile], sidx)
        pltpu.sync_copy(dst_hbm.at[tile], didx)

        def g_start(b, jj):
            pltpu.async_copy(tab.at[sidx.at[jj]], rows.at[b], gsem[b])

        def g_wait(b, jj):
            pltpu.make_async_copy(tab.at[sidx.at[jj]], rows.at[b],
                                  gsem[b]).wait()

        def s_start(b, jj):
            pltpu.async_copy(rows.at[b], acc.at[didx.at[jj]], ssem[b],
                             add=True)

        def s_wait(b, jj):
            pltpu.make_async_copy(rows.at[b], acc.at[didx.at[jj]],
                                  ssem[b]).wait()

        for b in range(nbuf):
            g_start(b, b)

        def it(i, _):
            for b in range(nbuf):
                g_wait(b, i * nbuf + b)
                s_start(b, i * nbuf + b)

            @pl.when(i < niter - 1)
            def _():
                for b in range(nbuf):
                    s_wait(b, i * nbuf + b)
                    g_start(b, (i + 1) * nbuf + b)

            return 0

        lax.fori_loop(0, niter, it, 0)
        for b in range(nbuf):
            s_wait(b, b)
        plsc.subcore_barrier()
        pltpu.sync_copy(acc.at[pl.ds(sid * NSLICE, NSLICE)],
                        out_hbm.at[cid, pl.ds(sid * NSLICE, NSLICE)])

    if fuse_deg:
        out_type = [
            jax.ShapeDtypeStruct((2, NPAD, width), jnp.float32),
            jax.ShapeDtypeStruct((2, NPAD), jnp.float32),
        ]
        scratch = [
            pltpu.VMEM((EB, LB), jnp.int32),
            pltpu.VMEM((EB, LB), jnp.int32),
            pltpu.VMEM((nbuf, LB, width), jnp.float32),
            pltpu.VMEM((LB, width), jnp.float32),
            pltpu.VMEM((NSLICE,), jnp.float32),
            pltpu.VMEM((NSLICE,), jnp.float32),
            pltpu.VMEM((NSLICE,), jnp.float32),
            pltpu.VMEM((LB, width), jnp.float32),
            pltpu.VMEM_SHARED((NPAD, width), jnp.float32),
            pltpu.VMEM_SHARED((NPAD, width), jnp.float32),
        ] + [pltpu.SemaphoreType.DMA] * (2 * nbuf)
    else:
        out_type = jax.ShapeDtypeStruct((2, NPAD, width), jnp.float32)
        scratch = [
            pltpu.VMEM((EB, LB), jnp.int32),
            pltpu.VMEM((EB, LB), jnp.int32),
            pltpu.VMEM((nbuf, LB, width), jnp.float32),
            pltpu.VMEM((LB, width), jnp.float32),
        ] + ([pltpu.VMEM_SHARED((NPAD, width), jnp.float32)]
             if stage_tab else []) + [
            pltpu.VMEM_SHARED((NPAD, width), jnp.float32),
        ] + [pltpu.SemaphoreType.DMA] * (2 * nbuf)
    return functools.partial(
        pl.kernel,
        out_type=out_type,
        mesh=_mesh(),
        scratch_types=scratch,
        compiler_params=_SC_PARAMS,
    )(body)


def _tc_b(x_ref, w1_ref, h_ref):
    h_ref[...] = jnp.dot(x_ref[...], w1_ref[...],
                         preferred_element_type=jnp.float32)


def _tc_d(p_ref, h1_ref, dinv_ref, b1_ref, w2_ref, gs_ref):
    p = p_ref[...]
    dinv = dinv_ref[...]
    s = p[0] + p[1] + h1_ref[...] * dinv
    out1 = s * dinv + b1_ref[...]
    h2 = jnp.maximum(out1, 0.0)
    g = jnp.dot(h2, w2_ref[...], preferred_element_type=jnp.float32)
    gs_ref[...] = g * dinv


def _tc_f(q_ref, gs_ref, dinv_ref, b2_ref, out_ref):
    q = q_ref[...]
    out2 = (q[0] + q[1] + gs_ref[...]) * dinv_ref[...] + b2_ref[...]
    a = out2[:, 0:1]
    b = out2[:, 1:2]
    m = jnp.maximum(a, b)
    lse = m + jnp.log(jnp.exp(a - m) + jnp.exp(b - m))
    out_ref[...] = out2[:, 0:2] - lse


def kernel(x, edge_index, W1, b1, W2, b2):
    src = edge_index[0].astype(jnp.int32)
    dst = edge_index[1].astype(jnp.int32)
    pad = jnp.full((EPAD - E,), NPAD - 1, jnp.int32)
    src3 = jnp.concatenate([src, pad]).reshape(NW, EB, LB)
    dst3 = jnp.concatenate([dst, pad]).reshape(NW, EB, LB)
    xp = jnp.pad(x, ((0, NPAD - N), (0, 0)))
    W1p = jnp.pad(W1, ((0, 0), (0, DHP - DH)))
    b1p = jnp.pad(b1, (0, DHP - DH)).reshape(1, DHP)
    W2p = jnp.pad(W2, ((0, DHP - DH), (0, DOP - DO)))
    b2p = jnp.pad(b2, (0, DOP - DO)).reshape(1, DOP)

    degp = _make_deg()(dst3)
    grid = NPAD // RB
    h1 = pl.pallas_call(
        _tc_b,
        grid=(grid,),
        in_specs=[
            pl.BlockSpec((RB, D_IN), lambda i: (i, 0)),
            pl.BlockSpec((D_IN, DHP), lambda i: (0, 0)),
        ],
        out_specs=pl.BlockSpec((RB, DHP), lambda i: (i, 0)),
        out_shape=jax.ShapeDtypeStruct((NPAD, DHP), jnp.float32),
    )(xp, W1p)

    p1, dinv2 = _make_msg(DHP, True, nbuf=5)(h1, src3, dst3, degp)
    dinv = dinv2[0].reshape(NPAD, 1)

    gs = pl.pallas_call(
        _tc_d,
        grid=(grid,),
        in_specs=[
            pl.BlockSpec((2, RB, DHP), lambda i: (0, i, 0)),
            pl.BlockSpec((RB, DHP), lambda i: (i, 0)),
            pl.BlockSpec((RB, 1), lambda i: (i, 0)),
            pl.BlockSpec((1, DHP), lambda i: (0, 0)),
            pl.BlockSpec((DHP, DOP), lambda i: (0, 0)),
        ],
        out_specs=pl.BlockSpec((RB, DOP), lambda i: (i, 0)),
        out_shape=jax.ShapeDtypeStruct((NPAD, DOP), jnp.float32),
    )(p1, h1, dinv, b1p, W2p)

    p2 = _make_msg(DOP, False, stage_tab=True, nbuf=8)(gs, src3, dst3)

    out = pl.pallas_call(
        _tc_f,
        grid=(grid,),
        in_specs=[
            pl.BlockSpec((2, RB, DOP), lambda i: (0, i, 0)),
            pl.BlockSpec((RB, DOP), lambda i: (i, 0)),
            pl.BlockSpec((RB, 1), lambda i: (i, 0)),
            pl.BlockSpec((1, DOP), lambda i: (0, 0)),
        ],
        out_specs=pl.BlockSpec((RB, DO), lambda i: (i, 0)),
        out_shape=jax.ShapeDtypeStruct((NPAD, DO), jnp.float32),
    )(p2, gs, dinv, b2p)

    return out[:N]
